# baseline (speedup 1.0000x reference)
import functools

import jax
import jax.numpy as jnp
from jax import lax
from jax.experimental import pallas as pl
from jax.experimental.pallas import tpu as pltpu
from jax.experimental.pallas import tpu_sc as plsc

N_NODES = 50000
N_EDGES = 800000
DIM_INNER = 128
DIM_EDGE = 16
NUM_EDGE_TYPES = 1000
EPS = 1e-5

NW = 32
NODE_CHUNK = 112
NODE_CHUNKS = 14
NODE_PER_W = NODE_CHUNK * NODE_CHUNKS
NBUF = 4
EDGE_PER_W = 25088
EDGE_TAB_PAD = 1024
EDGE_CHUNK = 1792
EDGE_CHUNKS = 14
HIST_GROUPS = EDGE_PER_W // 16
HIST_PER_CHUNK = HIST_GROUPS // NODE_CHUNKS

ROWS_BLK = 2000
N_BLKS = N_NODES // ROWS_BLK

_mesh = plsc.VectorSubcoreMesh(core_axis_name="c", subcore_axis_name="s")
_sc_params = pltpu.CompilerParams(needs_layout_passes=False,
                                  use_tc_tiling_on_sc=False)


@functools.partial(
    pl.kernel,
    mesh=_mesh,
    out_type=[
        jax.ShapeDtypeStruct((N_NODES, DIM_INNER), jnp.float32),
        jax.ShapeDtypeStruct((NW, EDGE_TAB_PAD // 16, 16), jnp.float32),
    ],
    scratch_types=[
        pltpu.VMEM((NODE_PER_W,), jnp.int32),
        pltpu.VMEM((NODE_CHUNK, DIM_INNER), jnp.float32),
        pltpu.VMEM((NODE_CHUNK, DIM_INNER), jnp.float32),
        pltpu.VMEM((NODE_CHUNK, DIM_INNER), jnp.float32),
        pltpu.VMEM((NODE_CHUNK, DIM_INNER), jnp.float32),
        pltpu.VMEM((EDGE_PER_W,), jnp.int32),
        pltpu.VMEM((EDGE_TAB_PAD // 16, 16), jnp.float32),
        pltpu.SemaphoreType.DMA,
        pltpu.SemaphoreType.DMA,
        pltpu.SemaphoreType.DMA,
        pltpu.SemaphoreType.DMA,
        pltpu.SemaphoreType.DMA,
        pltpu.SemaphoreType.DMA,
        pltpu.SemaphoreType.DMA,
        pltpu.SemaphoreType.DMA,
    ],
    compiler_params=_sc_params,
)
def _k1(x_hbm, eidx_hbm, tab_hbm, rawh_hbm, cnt_hbm,
        nidx_v, rb0, rb1, rb2, rb3, eidx_v, cnt_v,
        gs0, gs1, gs2, gs3, ws0, ws1, ws2, ws3):
    wid = lax.axis_index("s") * 2 + lax.axis_index("c")
    nbase = jnp.minimum(wid * NODE_PER_W, N_NODES - NODE_PER_W)
    ebase = jnp.minimum(wid * EDGE_PER_W, N_EDGES - EDGE_PER_W)
    ethr = wid * EDGE_PER_W - ebase

    bufs = (rb0, rb1, rb2, rb3)
    gsems = (gs0, gs1, gs2, gs3)
    wsems = (ws0, ws1, ws2, ws3)

    pltpu.sync_copy(x_hbm.at[pl.ds(nbase, NODE_PER_W)], nidx_v)
    pltpu.sync_copy(eidx_hbm.at[pl.ds(ebase, EDGE_PER_W)], eidx_v)

    zero16 = jnp.zeros((16,), jnp.float32)
    for i in range(EDGE_TAB_PAD // 16):
        cnt_v[i, pl.ds(0, 16)] = zero16

    iota = lax.iota(jnp.int32, 16)
    ethr16 = jnp.full((16,), 0, jnp.int32) + ethr

    g = [None] * NODE_CHUNKS
    w = [None] * NODE_CHUNKS
    for c in range(NBUF):
        g[c] = pltpu.async_copy(
            tab_hbm.at[nidx_v.at[pl.ds(c * NODE_CHUNK, NODE_CHUNK)]],
            bufs[c], gsems[c])

    def hist_body(i, carry):
        iv = eidx_v[pl.ds(i * 16, 16)]
        lpos = i * 16 + iota
        ones = jnp.where(lpos >= ethr16, 1.0, 0.0)
        plsc.addupdate_scatter(cnt_v, [iv >> 4, iv & 15], ones)
        return carry

    for c in range(NODE_CHUNKS):
        lax.fori_loop(c * HIST_PER_CHUNK, (c + 1) * HIST_PER_CHUNK,
                      hist_body, 0)

        bi = c % NBUF
        g[c].wait()
        w[c] = pltpu.async_copy(
            bufs[bi], rawh_hbm.at[pl.ds(nbase + c * NODE_CHUNK, NODE_CHUNK)],
            wsems[bi])
        n = c + NBUF
        if n < NODE_CHUNKS:
            w[c].wait()
            g[n] = pltpu.async_copy(
                tab_hbm.at[nidx_v.at[pl.ds(n * NODE_CHUNK, NODE_CHUNK)]],
                bufs[bi], gsems[bi])

    for c in range(NODE_CHUNKS - NBUF, NODE_CHUNKS):
        w[c].wait()
    pltpu.sync_copy(cnt_v, cnt_hbm.at[wid])


def _k2n_body(rawh, ng, nb, nscale, nshift, acc):
    i = pl.program_id(0)

    @pl.when(i == 0)
    def _init():
        acc[...] = jnp.zeros((2, DIM_INNER), jnp.float32)

    blk = rawh[...]
    acc[0:1, :] += jnp.sum(blk, axis=0, keepdims=True)
    acc[1:2, :] += jnp.sum(blk * blk, axis=0, keepdims=True)

    @pl.when(i == N_BLKS - 1)
    def _fin():
        mean = acc[0:1, :] / N_NODES
        var = acc[1:2, :] / N_NODES - mean * mean
        inv = lax.rsqrt(var + EPS)
        sc = ng[...][None, :] * inv
        nscale[...] = sc
        nshift[...] = nb[...][None, :] - mean * sc


def _k2e_body(cnt, etab, eg, eb, etabn):
    crow = jnp.sum(cnt[...], axis=0, keepdims=True)
    t = etab[...]
    esum = jnp.dot(crow, t, preferred_element_type=jnp.float32)
    esq = jnp.dot(crow, t * t, preferred_element_type=jnp.float32)
    em = esum / N_EDGES
    ev = esq / N_EDGES - em * em
    einv = lax.rsqrt(ev + EPS)
    esc = eg[...][None, :] * einv
    esh = eb[...][None, :] - em * esc
    etabn[...] = (t * esc + esh).T


def _k3a_body(raw, scale, shift, out):
    out[...] = raw[...] * scale[...] + shift[...]


HALF_ROWS = N_EDGES // 128 * 8
CHUNK_ROWS = EDGE_CHUNK // 128 * 8


CHUNK_ELEMS = CHUNK_ROWS * DIM_INNER


@functools.partial(
    pl.kernel,
    mesh=_mesh,
    out_type=jax.ShapeDtypeStruct((2 * HALF_ROWS * DIM_INNER,), jnp.float32),
    scratch_types=[
        pltpu.VMEM((EDGE_TAB_PAD * DIM_EDGE,), jnp.float32),
        pltpu.VMEM((EDGE_PER_W,), jnp.int32),
        pltpu.VMEM((2 * CHUNK_ELEMS,), jnp.float32),
        pltpu.VMEM((2 * CHUNK_ELEMS,), jnp.float32),
        pltpu.SemaphoreType.DMA,
        pltpu.SemaphoreType.DMA,
        pltpu.SemaphoreType.DMA,
        pltpu.SemaphoreType.DMA,
    ],
    compiler_params=_sc_params,
)
def _k3b(eidx_hbm, etabn_hbm, out_hbm, tab_v, eidx_v, ob0, ob1,
         os0, os1, os2, os3):
    wid = lax.axis_index("s") * 2 + lax.axis_index("c")
    base = jnp.minimum(wid * EDGE_PER_W, N_EDGES - EDGE_PER_W)
    rbase = base // 128 * 8
    pltpu.sync_copy(etabn_hbm, tab_v)
    pltpu.sync_copy(eidx_hbm.at[pl.ds(base, EDGE_PER_W)], eidx_v)

    obufs = (ob0, ob1)
    osems = ((os0, os1), (os2, os3))
    foff = [(112 * (f // 8) + (f % 8)) * DIM_INNER for f in range(DIM_EDGE)]
    w = [[None, None] for _ in range(EDGE_CHUNKS)]
    for c in range(EDGE_CHUNKS):
        bi = c % 2
        out_v = obufs[bi]

        def edge_body(i0, carry):
            for j in range(8):
                i = i0 * 8 + j
                iv = eidx_v[pl.ds(c * EDGE_CHUNK + i * 16, 16)]
                off = ((i >> 3) << 10) + ((i & 7) << 4)
                for f in range(DIM_EDGE):
                    vals = plsc.load_gather(
                        tab_v.at[pl.ds(f * EDGE_TAB_PAD, EDGE_TAB_PAD)], [iv])
                    out_v[pl.ds(off + foff[f], 16)] = vals
            return carry

        if c >= 2:
            w[c - 2][0].wait()
            w[c - 2][1].wait()
        lax.fori_loop(0, EDGE_CHUNK // 128, edge_body, 0)
        eb = (rbase + c * CHUNK_ROWS) * DIM_INNER
        w[c][0] = pltpu.async_copy(
            out_v.at[pl.ds(0, CHUNK_ELEMS)],
            out_hbm.at[pl.ds(eb, CHUNK_ELEMS)], osems[0][bi])
        w[c][1] = pltpu.async_copy(
            out_v.at[pl.ds(CHUNK_ELEMS, CHUNK_ELEMS)],
            out_hbm.at[pl.ds(HALF_ROWS * DIM_INNER + eb, CHUNK_ELEMS)],
            osems[1][bi])
    for c in (EDGE_CHUNKS - 2, EDGE_CHUNKS - 1):
        w[c][0].wait()
        w[c][1].wait()


def kernel(x, edge_attr, node_table, edge_table, node_gamma, node_beta,
           edge_gamma, edge_beta):
    etab_pad = jnp.concatenate(
        [edge_table,
         jnp.zeros((EDGE_TAB_PAD - NUM_EDGE_TYPES, DIM_EDGE),
                   edge_table.dtype)])

    rawh, cnt = _k1(x, edge_attr, node_table)
    cnt = cnt.reshape(NW, EDGE_TAB_PAD)

    etabn = pl.pallas_call(
        _k2e_body,
        out_shape=jax.ShapeDtypeStruct((DIM_EDGE, EDGE_TAB_PAD), jnp.float32),
    )(cnt, etab_pad, edge_gamma, edge_beta)

    e_packed = _k3b(edge_attr, etabn.reshape(-1))

    nscale, nshift = pl.pallas_call(
        _k2n_body,
        grid=(N_BLKS,),
        in_specs=[
            pl.BlockSpec((ROWS_BLK, DIM_INNER), lambda i: (i, 0)),
            pl.BlockSpec((DIM_INNER,), lambda i: (0,)),
            pl.BlockSpec((DIM_INNER,), lambda i: (0,)),
        ],
        out_specs=[
            pl.BlockSpec((1, DIM_INNER), lambda i: (0, 0)),
            pl.BlockSpec((1, DIM_INNER), lambda i: (0, 0)),
        ],
        out_shape=[
            jax.ShapeDtypeStruct((1, DIM_INNER), jnp.float32),
            jax.ShapeDtypeStruct((1, DIM_INNER), jnp.float32),
        ],
        scratch_shapes=[pltpu.VMEM((2, DIM_INNER), jnp.float32)],
    )(rawh, node_gamma, node_beta)

    h = pl.pallas_call(
        _k3a_body,
        grid=(N_BLKS,),
        in_specs=[
            pl.BlockSpec((ROWS_BLK, DIM_INNER), lambda i: (i, 0)),
            pl.BlockSpec((1, DIM_INNER), lambda i: (0, 0)),
            pl.BlockSpec((1, DIM_INNER), lambda i: (0, 0)),
        ],
        out_specs=pl.BlockSpec((ROWS_BLK, DIM_INNER), lambda i: (i, 0)),
        out_shape=jax.ShapeDtypeStruct((N_NODES, DIM_INNER), jnp.float32),
    )(rawh, nscale, nshift)

    e_out = e_packed.reshape(2, N_EDGES // 128, 8, 128) \
        .transpose(1, 3, 0, 2).reshape(N_EDGES, DIM_EDGE)
    return h, e_out


# speedup vs baseline: 1.0067x; 1.0067x over previous
import functools

import jax
import jax.numpy as jnp
from jax import lax
from jax.experimental import pallas as pl
from jax.experimental.pallas import tpu as pltpu
from jax.experimental.pallas import tpu_sc as plsc

N_NODES = 50000
N_EDGES = 800000
DIM_INNER = 128
DIM_EDGE = 16
NUM_EDGE_TYPES = 1000
EPS = 1e-5

NW = 32
NODE_CHUNK = 112
NODE_CHUNKS = 14
NODE_PER_W = NODE_CHUNK * NODE_CHUNKS
NBUF = 4
EDGE_PER_W = 25088
EDGE_TAB_PAD = 1024
EDGE_CHUNK = 1792
EDGE_CHUNKS = 14
HIST_GROUPS = EDGE_PER_W // 16
HIST_PER_CHUNK = HIST_GROUPS // NODE_CHUNKS

ROWS_BLK = 2000
N_BLKS = N_NODES // ROWS_BLK

_mesh = plsc.VectorSubcoreMesh(core_axis_name="c", subcore_axis_name="s")
_sc_params = pltpu.CompilerParams(needs_layout_passes=False,
                                  use_tc_tiling_on_sc=False)


@functools.partial(
    pl.kernel,
    mesh=_mesh,
    out_type=[
        jax.ShapeDtypeStruct((N_NODES, DIM_INNER), jnp.float32),
        jax.ShapeDtypeStruct((NW, EDGE_TAB_PAD // 16, 16), jnp.float32),
    ],
    scratch_types=[
        pltpu.VMEM((NODE_PER_W,), jnp.int32),
        pltpu.VMEM((NODE_CHUNK, DIM_INNER), jnp.float32),
        pltpu.VMEM((NODE_CHUNK, DIM_INNER), jnp.float32),
        pltpu.VMEM((NODE_CHUNK, DIM_INNER), jnp.float32),
        pltpu.VMEM((NODE_CHUNK, DIM_INNER), jnp.float32),
        pltpu.VMEM((EDGE_PER_W,), jnp.int32),
        pltpu.VMEM((EDGE_TAB_PAD // 16, 16), jnp.float32),
        pltpu.SemaphoreType.DMA,
        pltpu.SemaphoreType.DMA,
        pltpu.SemaphoreType.DMA,
        pltpu.SemaphoreType.DMA,
        pltpu.SemaphoreType.DMA,
        pltpu.SemaphoreType.DMA,
        pltpu.SemaphoreType.DMA,
        pltpu.SemaphoreType.DMA,
    ],
    compiler_params=_sc_params,
)
def _k1(x_hbm, eidx_hbm, tab_hbm, rawh_hbm, cnt_hbm,
        nidx_v, rb0, rb1, rb2, rb3, eidx_v, cnt_v,
        gs0, gs1, gs2, gs3, ws0, ws1, ws2, ws3):
    wid = lax.axis_index("s") * 2 + lax.axis_index("c")
    nbase = jnp.minimum(wid * NODE_PER_W, N_NODES - NODE_PER_W)
    ebase = jnp.minimum(wid * EDGE_PER_W, N_EDGES - EDGE_PER_W)
    ethr = wid * EDGE_PER_W - ebase

    bufs = (rb0, rb1, rb2, rb3)
    gsems = (gs0, gs1, gs2, gs3)
    wsems = (ws0, ws1, ws2, ws3)

    pltpu.sync_copy(x_hbm.at[pl.ds(nbase, NODE_PER_W)], nidx_v)
    pltpu.sync_copy(eidx_hbm.at[pl.ds(ebase, EDGE_PER_W)], eidx_v)

    zero16 = jnp.zeros((16,), jnp.float32)
    for i in range(EDGE_TAB_PAD // 16):
        cnt_v[i, pl.ds(0, 16)] = zero16

    iota = lax.iota(jnp.int32, 16)
    ethr16 = jnp.full((16,), 0, jnp.int32) + ethr

    g = [None] * NODE_CHUNKS
    w = [None] * NODE_CHUNKS
    for c in range(NBUF):
        g[c] = pltpu.async_copy(
            tab_hbm.at[nidx_v.at[pl.ds(c * NODE_CHUNK, NODE_CHUNK)]],
            bufs[c], gsems[c])

    def hist_body(i, carry):
        iv = eidx_v[pl.ds(i * 16, 16)]
        lpos = i * 16 + iota
        ones = jnp.where(lpos >= ethr16, 1.0, 0.0)
        plsc.addupdate_scatter(cnt_v, [iv >> 4, iv & 15], ones)
        return carry

    for c in range(NODE_CHUNKS):
        lax.fori_loop(c * HIST_PER_CHUNK, (c + 1) * HIST_PER_CHUNK,
                      hist_body, 0)

        bi = c % NBUF
        g[c].wait()
        w[c] = pltpu.async_copy(
            bufs[bi], rawh_hbm.at[pl.ds(nbase + c * NODE_CHUNK, NODE_CHUNK)],
            wsems[bi])
        n = c + NBUF
        if n < NODE_CHUNKS:
            w[c].wait()
            g[n] = pltpu.async_copy(
                tab_hbm.at[nidx_v.at[pl.ds(n * NODE_CHUNK, NODE_CHUNK)]],
                bufs[bi], gsems[bi])

    for c in range(NODE_CHUNKS - NBUF, NODE_CHUNKS):
        w[c].wait()
    pltpu.sync_copy(cnt_v, cnt_hbm.at[wid])


def _k2n_body(rawh, ng, nb, nscale, nshift, acc):
    i = pl.program_id(0)

    @pl.when(i == 0)
    def _init():
        acc[...] = jnp.zeros((2, DIM_INNER), jnp.float32)

    blk = rawh[...]
    acc[0:1, :] += jnp.sum(blk, axis=0, keepdims=True)
    acc[1:2, :] += jnp.sum(blk * blk, axis=0, keepdims=True)

    @pl.when(i == N_BLKS - 1)
    def _fin():
        mean = acc[0:1, :] / N_NODES
        var = acc[1:2, :] / N_NODES - mean * mean
        inv = lax.rsqrt(var + EPS)
        sc = ng[...][None, :] * inv
        nscale[...] = sc
        nshift[...] = nb[...][None, :] - mean * sc


def _k2e_body(cnt, etab, eg, eb, etabn):
    crow = jnp.sum(cnt[...], axis=0, keepdims=True)
    t = etab[...]
    esum = jnp.dot(crow, t, preferred_element_type=jnp.float32)
    esq = jnp.dot(crow, t * t, preferred_element_type=jnp.float32)
    em = esum / N_EDGES
    ev = esq / N_EDGES - em * em
    einv = lax.rsqrt(ev + EPS)
    esc = eg[...][None, :] * einv
    esh = eb[...][None, :] - em * esc
    etabn[...] = (t * esc + esh).T


def _k3a_body(raw, scale, shift, out):
    out[...] = raw[...] * scale[...] + shift[...]


HALF_ROWS = N_EDGES // 128 * 8
CHUNK_ROWS = EDGE_CHUNK // 128 * 8


CHUNK_ELEMS = CHUNK_ROWS * DIM_INNER


@functools.partial(
    pl.kernel,
    mesh=_mesh,
    out_type=jax.ShapeDtypeStruct((2 * HALF_ROWS * DIM_INNER,), jnp.float32),
    scratch_types=[
        pltpu.VMEM((EDGE_TAB_PAD * DIM_EDGE,), jnp.float32),
        pltpu.VMEM((EDGE_PER_W,), jnp.int32),
        pltpu.VMEM((2 * CHUNK_ELEMS,), jnp.float32),
        pltpu.VMEM((2 * CHUNK_ELEMS,), jnp.float32),
        pltpu.SemaphoreType.DMA,
        pltpu.SemaphoreType.DMA,
        pltpu.SemaphoreType.DMA,
        pltpu.SemaphoreType.DMA,
    ],
    compiler_params=_sc_params,
)
def _k3b(eidx_hbm, etabn_hbm, out_hbm, tab_v, eidx_v, ob0, ob1,
         os0, os1, os2, os3):
    wid = lax.axis_index("s") * 2 + lax.axis_index("c")
    base = jnp.minimum(wid * EDGE_PER_W, N_EDGES - EDGE_PER_W)
    rbase = base // 128 * 8
    pltpu.sync_copy(etabn_hbm, tab_v)
    pltpu.sync_copy(eidx_hbm.at[pl.ds(base, EDGE_PER_W)], eidx_v)

    obufs = (ob0, ob1)
    osems = ((os0, os1), (os2, os3))
    foff = [(112 * (f // 8) + (f % 8)) * DIM_INNER for f in range(DIM_EDGE)]
    w = [[None, None] for _ in range(EDGE_CHUNKS)]
    for c in range(EDGE_CHUNKS):
        bi = c % 2
        out_v = obufs[bi]

        def edge_body(i0, carry):
            for j in range(4):
                i = i0 * 4 + j
                iv = eidx_v[pl.ds(c * EDGE_CHUNK + i * 16, 16)]
                off = ((i >> 3) << 10) + ((i & 7) << 4)
                for f in range(DIM_EDGE):
                    vals = plsc.load_gather(
                        tab_v.at[pl.ds(f * EDGE_TAB_PAD, EDGE_TAB_PAD)], [iv])
                    out_v[pl.ds(off + foff[f], 16)] = vals
            return carry

        if c >= 2:
            w[c - 2][0].wait()
            w[c - 2][1].wait()
        lax.fori_loop(0, EDGE_CHUNK // 64, edge_body, 0)
        eb = (rbase + c * CHUNK_ROWS) * DIM_INNER
        w[c][0] = pltpu.async_copy(
            out_v.at[pl.ds(0, CHUNK_ELEMS)],
            out_hbm.at[pl.ds(eb, CHUNK_ELEMS)], osems[0][bi])
        w[c][1] = pltpu.async_copy(
            out_v.at[pl.ds(CHUNK_ELEMS, CHUNK_ELEMS)],
            out_hbm.at[pl.ds(HALF_ROWS * DIM_INNER + eb, CHUNK_ELEMS)],
            osems[1][bi])
    for c in (EDGE_CHUNKS - 2, EDGE_CHUNKS - 1):
        w[c][0].wait()
        w[c][1].wait()


def kernel(x, edge_attr, node_table, edge_table, node_gamma, node_beta,
           edge_gamma, edge_beta):
    etab_pad = jnp.concatenate(
        [edge_table,
         jnp.zeros((EDGE_TAB_PAD - NUM_EDGE_TYPES, DIM_EDGE),
                   edge_table.dtype)])

    rawh, cnt = _k1(x, edge_attr, node_table)
    cnt = cnt.reshape(NW, EDGE_TAB_PAD)

    etabn = pl.pallas_call(
        _k2e_body,
        out_shape=jax.ShapeDtypeStruct((DIM_EDGE, EDGE_TAB_PAD), jnp.float32),
    )(cnt, etab_pad, edge_gamma, edge_beta)

    e_packed = _k3b(edge_attr, etabn.reshape(-1))

    nscale, nshift = pl.pallas_call(
        _k2n_body,
        grid=(N_BLKS,),
        in_specs=[
            pl.BlockSpec((ROWS_BLK, DIM_INNER), lambda i: (i, 0)),
            pl.BlockSpec((DIM_INNER,), lambda i: (0,)),
            pl.BlockSpec((DIM_INNER,), lambda i: (0,)),
        ],
        out_specs=[
            pl.BlockSpec((1, DIM_INNER), lambda i: (0, 0)),
            pl.BlockSpec((1, DIM_INNER), lambda i: (0, 0)),
        ],
        out_shape=[
            jax.ShapeDtypeStruct((1, DIM_INNER), jnp.float32),
            jax.ShapeDtypeStruct((1, DIM_INNER), jnp.float32),
        ],
        scratch_shapes=[pltpu.VMEM((2, DIM_INNER), jnp.float32)],
    )(rawh, node_gamma, node_beta)

    h = pl.pallas_call(
        _k3a_body,
        grid=(N_BLKS,),
        in_specs=[
            pl.BlockSpec((ROWS_BLK, DIM_INNER), lambda i: (i, 0)),
            pl.BlockSpec((1, DIM_INNER), lambda i: (0, 0)),
            pl.BlockSpec((1, DIM_INNER), lambda i: (0, 0)),
        ],
        out_specs=pl.BlockSpec((ROWS_BLK, DIM_INNER), lambda i: (i, 0)),
        out_shape=jax.ShapeDtypeStruct((N_NODES, DIM_INNER), jnp.float32),
    )(rawh, nscale, nshift)

    e_out = e_packed.reshape(2, N_EDGES // 128, 8, 128) \
        .transpose(1, 3, 0, 2).reshape(N_EDGES, DIM_EDGE)
    return h, e_out
